# Initial kernel scaffold; baseline (speedup 1.0000x reference)
import jax
import jax.numpy as jnp
from jax.experimental import pallas as pl


def kernel(query, key, value, top_k_centers):
    raise NotImplementedError("write your pallas kernel here")


# speedup vs baseline: 1.8923x; 1.8923x over previous
import jax
import jax.numpy as jnp
from jax import lax
from jax.experimental import pallas as pl
from jax.experimental.pallas import tpu as pltpu

_N = 900


def _pairs_body(c_ref, ct_ref, out_ref):
    col = c_ref[0]
    ct = ct_ref[0]

    cx_c, cy_c = col[:, 0:1], col[:, 1:2]
    h_c, w_c = col[:, 2:3], col[:, 3:4]
    x1_c = cx_c - 0.5 * w_c
    y1_c = cy_c - 0.5 * h_c
    x2_c = cx_c + 0.5 * w_c
    y2_c = cy_c + 0.5 * h_c

    cx_r, cy_r = ct[0:1, :], ct[1:2, :]
    h_r, w_r = ct[2:3, :], ct[3:4, :]
    x1_r = cx_r - 0.5 * w_r
    y1_r = cy_r - 0.5 * h_r
    x2_r = cx_r + 0.5 * w_r
    y2_r = cy_r + 0.5 * h_r

    iw = jnp.maximum(jnp.maximum(x2_c, x2_r) - jnp.maximum(x1_c, x1_r), 0.0)
    ih = jnp.maximum(jnp.maximum(y2_c, y2_r) - jnp.maximum(y1_c, y1_r), 0.0)
    inter = iw * ih

    area_c = (x2_c - x1_c) * (y2_c - y1_c)
    area_r = (x2_r - x1_r) * (y2_r - y1_r)
    union = area_c + area_r - inter

    ii = lax.broadcasted_iota(jnp.int32, (_N, _N), 0)
    jj = lax.broadcasted_iota(jnp.int32, (_N, _N), 1)
    iou = inter / union - jnp.where(ii == jj, 1.0, 0.0)

    m = jnp.max(iou, axis=1, keepdims=True)
    am = jnp.min(jnp.where(iou == m, jj, _N), axis=1, keepdims=True)
    am = jnp.minimum(am, _N - 1)

    l1_c = jnp.abs(x2_c - x1_c) + jnp.abs(y2_c - y1_c)
    l1_r = jnp.abs(x2_r - x1_r) + jnp.abs(y2_r - y1_r)
    gathered = jnp.max(jnp.where(jj == am, jnp.broadcast_to(l1_r, (_N, _N)), 0.0),
                       axis=1, keepdims=True)

    idx0 = lax.broadcasted_iota(jnp.int32, (_N, 1), 0)
    keep = l1_c >= gathered
    out0 = jnp.where(keep, idx0, am)
    out1 = jnp.where(keep, am, idx0)
    out_ref[0] = jnp.concatenate([out0, out1], axis=1)


def kernel(query, key, value, top_k_centers):
    del query, key, value
    b = top_k_centers.shape[0]
    ct = jnp.transpose(top_k_centers, (0, 2, 1))
    return pl.pallas_call(
        _pairs_body,
        grid=(b,),
        in_specs=[
            pl.BlockSpec((1, _N, 4), lambda i: (i, 0, 0)),
            pl.BlockSpec((1, 4, _N), lambda i: (i, 0, 0)),
        ],
        out_specs=pl.BlockSpec((1, _N, 2), lambda i: (i, 0, 0)),
        out_shape=jax.ShapeDtypeStruct((b, _N, 2), jnp.int32),
    )(top_k_centers, ct)
